# baseline (speedup 1.0000x reference)
import functools

import jax
import jax.numpy as jnp
import numpy as np
from jax.experimental import pallas as pl

MAX_STEPS_ = 1000
ROWS_PER_BLOCK = 512
ROW_LEN = 2048

_U32 = jnp.uint32
_KS1 = 42
_KS2 = 0x1BD11BDA ^ 42
_ERFINV_LT = [3.43273939e-07, -3.5233877e-06, -4.39150654e-06, 0.00021858087,
              -0.00125372503, -0.00417768164, 0.246640727, 1.50140941]
_ERFINV_GT = [0.000100950558, 0.00134934322, -0.00367342844, 0.00573950773,
              -0.0076224613, 0.00943887047, 1.00167406, 2.83297682]


def _rotl(x, r):
    return jax.lax.shift_left(x, _U32(r)) | jax.lax.shift_right_logical(x, _U32(32 - r))


def _threefry_bits(idx):
    x1 = idx + _U32(_KS1)
    x0 = x1
    x1 = x0 ^ _rotl(x1, 13)
    for r in (15, 26, 6):
        x0 = x0 + x1
        x1 = x0 ^ _rotl(x1, r)
    x0 = x0 + _U32(_KS1)
    x1 = x1 + _U32(_KS2 + 1)
    for g, rots in ((1, (17, 29, 16, 24)), (2, (13, 15, 26, 6)),
                    (3, (17, 29, 16, 24)), (4, (13, 15, 26, 6))):
        for r in rots:
            x0 = x0 + x1
            x1 = x0 ^ _rotl(x1, r)
        ks = (0, _KS1, _KS2)
        x0 = x0 + _U32(ks[(g + 1) % 3])
        x1 = x1 + _U32((ks[(g + 2) % 3] + g + 1) % (1 << 32))
    return x0 ^ x1


def _bits_to_normal(bits):
    lo = np.nextafter(np.float32(-1.0), np.float32(0.0))
    span = np.float32(1.0) - lo
    f = jax.lax.bitcast_convert_type(
        jax.lax.shift_right_logical(bits, _U32(9)) | _U32(0x3F800000), jnp.float32)
    u = jnp.maximum(jnp.float32(lo), (f - 1.0) * span + lo)
    w = -jnp.log1p(-u * u)
    w1 = w - 2.5
    p1 = jnp.float32(2.81022636e-08)
    for c in _ERFINV_LT:
        p1 = jnp.float32(c) + p1 * w1
    w2 = jnp.sqrt(w) - 3.0
    p2 = jnp.float32(-0.000200214257)
    for c in _ERFINV_GT:
        p2 = jnp.float32(c) + p2 * w2
    p = jnp.where(w < 5.0, p1, p2)
    return np.float32(np.sqrt(2.0)) * p * u


def _block_body(x_ref, c_ref, o_ref):
    i = pl.program_id(0)
    rows, cols = x_ref.shape
    base = (i * rows * cols).astype(_U32)
    idx = (base
           + jax.lax.broadcasted_iota(_U32, (rows, cols), 0) * _U32(cols)
           + jax.lax.broadcasted_iota(_U32, (rows, cols), 1))
    noise = _bits_to_normal(_threefry_bits(idx))
    x = x_ref[...]
    o_ref[...] = x + c_ref[...] * (noise - x)


@functools.partial(jax.jit, static_argnames=())
def kernel(X, steps, mask):
    b, s, d = X.shape
    n_rows = b * s
    ni = 1.0 - jnp.cos(jnp.pi * (1.0 - steps.astype(X.dtype) / MAX_STEPS_) / 2.0)
    coef = jnp.where(mask, (1.0 - ni)[:, None], 0.0).astype(X.dtype)
    coef = coef.reshape(n_rows, 1)
    x2 = X.reshape(n_rows, d)
    grid = n_rows // ROWS_PER_BLOCK
    out = pl.pallas_call(
        _block_body,
        grid=(grid,),
        in_specs=[
            pl.BlockSpec((ROWS_PER_BLOCK, d), lambda i: (i, 0)),
            pl.BlockSpec((ROWS_PER_BLOCK, 1), lambda i: (i, 0)),
        ],
        out_specs=pl.BlockSpec((ROWS_PER_BLOCK, d), lambda i: (i, 0)),
        out_shape=jax.ShapeDtypeStruct((n_rows, d), X.dtype),
    )(x2, coef)
    return out.reshape(b, s, d)


# speedup vs baseline: 1.0001x; 1.0001x over previous
import functools

import jax
import jax.numpy as jnp
import numpy as np
from jax.experimental import pallas as pl
from jax.experimental.pallas import tpu as pltpu

MAX_STEPS_ = 1000
ROWS_PER_BLOCK = 512
ROW_LEN = 2048

_U32 = jnp.uint32
_KS1 = 42
_KS2 = 0x1BD11BDA ^ 42
_ERFINV_LT = [3.43273939e-07, -3.5233877e-06, -4.39150654e-06, 0.00021858087,
              -0.00125372503, -0.00417768164, 0.246640727, 1.50140941]
_ERFINV_GT = [0.000100950558, 0.00134934322, -0.00367342844, 0.00573950773,
              -0.0076224613, 0.00943887047, 1.00167406, 2.83297682]


def _rotl(x, r):
    return jax.lax.shift_left(x, _U32(r)) | jax.lax.shift_right_logical(x, _U32(32 - r))


def _threefry_bits(idx):
    x1 = idx + _U32(_KS1)
    x0 = x1
    x1 = x0 ^ _rotl(x1, 13)
    for r in (15, 26, 6):
        x0 = x0 + x1
        x1 = x0 ^ _rotl(x1, r)
    x0 = x0 + _U32(_KS1)
    x1 = x1 + _U32(_KS2 + 1)
    for g, rots in ((1, (17, 29, 16, 24)), (2, (13, 15, 26, 6)),
                    (3, (17, 29, 16, 24)), (4, (13, 15, 26, 6))):
        for r in rots:
            x0 = x0 + x1
            x1 = x0 ^ _rotl(x1, r)
        ks = (0, _KS1, _KS2)
        x0 = x0 + _U32(ks[(g + 1) % 3])
        x1 = x1 + _U32((ks[(g + 2) % 3] + g + 1) % (1 << 32))
    return x0 ^ x1


def _bits_to_normal(bits):
    lo = np.nextafter(np.float32(-1.0), np.float32(0.0))
    span = np.float32(1.0) - lo
    f = jax.lax.bitcast_convert_type(
        jax.lax.shift_right_logical(bits, _U32(9)) | _U32(0x3F800000), jnp.float32)
    u = jnp.maximum(jnp.float32(lo), (f - 1.0) * span + lo)
    w = -jnp.log1p(-u * u)
    w1 = w - 2.5
    p1 = jnp.float32(2.81022636e-08)
    for c in _ERFINV_LT:
        p1 = jnp.float32(c) + p1 * w1
    w2 = jnp.sqrt(w) - 3.0
    p2 = jnp.float32(-0.000200214257)
    for c in _ERFINV_GT:
        p2 = jnp.float32(c) + p2 * w2
    p = jnp.where(w < 5.0, p1, p2)
    return np.float32(np.sqrt(2.0)) * p * u


def _block_body(x_ref, c_ref, o_ref):
    i = pl.program_id(0)
    rows, cols = x_ref.shape
    base = (i * rows * cols).astype(_U32)
    idx = (base
           + jax.lax.broadcasted_iota(_U32, (rows, cols), 0) * _U32(cols)
           + jax.lax.broadcasted_iota(_U32, (rows, cols), 1))
    noise = _bits_to_normal(_threefry_bits(idx))
    x = x_ref[...]
    o_ref[...] = x + c_ref[...] * (noise - x)


@functools.partial(jax.jit, static_argnames=())
def kernel(X, steps, mask):
    b, s, d = X.shape
    n_rows = b * s
    ni = 1.0 - jnp.cos(jnp.pi * (1.0 - steps.astype(X.dtype) / MAX_STEPS_) / 2.0)
    coef = jnp.where(mask, (1.0 - ni)[:, None], 0.0).astype(X.dtype)
    coef = coef.reshape(n_rows, 1)
    x2 = X.reshape(n_rows, d)
    grid = n_rows // ROWS_PER_BLOCK
    out = pl.pallas_call(
        _block_body,
        grid=(grid,),
        in_specs=[
            pl.BlockSpec((ROWS_PER_BLOCK, d), lambda i: (i, 0)),
            pl.BlockSpec((ROWS_PER_BLOCK, 1), lambda i: (i, 0)),
        ],
        out_specs=pl.BlockSpec((ROWS_PER_BLOCK, d), lambda i: (i, 0)),
        out_shape=jax.ShapeDtypeStruct((n_rows, d), X.dtype),
        compiler_params=pltpu.CompilerParams(
            dimension_semantics=("parallel",)),
    )(x2, coef)
    return out.reshape(b, s, d)


# speedup vs baseline: 2.3394x; 2.3392x over previous
import jax
import jax.numpy as jnp
import numpy as np
from jax.experimental import pallas as pl
from jax.experimental.pallas import tpu as pltpu

MAX_STEPS_ = 1000
ROWS_PER_BLOCK = 256
ROW_LEN = 2048
CHUNK_R = 8
CHUNK_C = 1024

_U32 = jnp.uint32
_KS1 = 42
_KS2 = 0x1BD11BDA ^ 42

_SQRT2 = np.sqrt(2.0)


def _expand(coefs_lowfirst, shift, scale):
    p = np.polynomial.Polynomial(coefs_lowfirst)
    q = p(np.polynomial.Polynomial([shift, scale]))
    return [np.float32(c) for c in q.coef]


_P1 = [1.50140941, 0.246640727, -0.00417768164, -0.00125372503,
       0.00021858087, -4.39150654e-06, -3.5233877e-06, 3.43273939e-07,
       2.81022636e-08]
_P2 = [2.83297682, 1.00167406, 0.00943887047, -0.0076224613, 0.00573950773,
       -0.00367342844, 0.00134934322, 0.000100950558, -0.000200214257]
_Q1 = _expand([_SQRT2 * c for c in _P1], -2.5, -1.0)
_Q2 = _expand([_SQRT2 * c for c in _P2], -3.0, 1.0)

_UNIF_LO = np.nextafter(np.float32(-1.0), np.float32(0.0))
_UNIF_SPAN = np.float32(np.float32(1.0) - _UNIF_LO)
_UNIF_OFF = np.float32(_UNIF_LO - _UNIF_SPAN)


def _rotl(x, r):
    return jax.lax.shift_left(x, _U32(r)) | jax.lax.shift_right_logical(x, _U32(32 - r))


def _threefry_bits(x1):
    x0 = x1
    x1 = x0 ^ _rotl(x1, 13)
    for r in (15, 26, 6):
        x0 = x0 + x1
        x1 = x0 ^ _rotl(x1, r)
    x0 = x0 + _U32(_KS1)
    x1 = x1 + _U32(_KS2 + 1)
    for g, rots in ((1, (17, 29, 16, 24)), (2, (13, 15, 26, 6)),
                    (3, (17, 29, 16, 24)), (4, (13, 15, 26, 6))):
        for r in rots:
            x0 = x0 + x1
            x1 = x0 ^ _rotl(x1, r)
        ks = (0, _KS1, _KS2)
        x0 = x0 + _U32(ks[(g + 1) % 3])
        x1 = x1 + _U32((ks[(g + 2) % 3] + g + 1) % (1 << 32))
    return x0 ^ x1


def _bits_to_normal(bits):
    f = jax.lax.bitcast_convert_type(
        jax.lax.shift_right_logical(bits, _U32(9)) | _U32(0x3F800000), jnp.float32)
    u = f * _UNIF_SPAN + _UNIF_OFF
    u = jnp.clip(u, _UNIF_LO, -_UNIF_LO)
    s = 1.0 - u * u
    el = jnp.log(s)
    p1 = jnp.float32(_Q1[-1])
    for c in _Q1[-2::-1]:
        p1 = c + p1 * el
    sq = jnp.sqrt(-el)
    p2 = jnp.float32(_Q2[-1])
    for c in _Q2[-2::-1]:
        p2 = c + p2 * sq
    p = jnp.where(el > -5.0, p1, p2)
    return p * u


def _block_body(x_ref, c_ref, o_ref):
    i = pl.program_id(0)
    rows, cols = x_ref.shape
    nc = cols // CHUNK_C
    nchunks = (rows // CHUNK_R) * nc
    iota = (jax.lax.broadcasted_iota(_U32, (CHUNK_R, CHUNK_C), 0) * _U32(cols)
            + jax.lax.broadcasted_iota(_U32, (CHUNK_R, CHUNK_C), 1)
            + _U32(_KS1))
    block_base = i * rows * cols

    def body(k, carry):
        r = (k // nc) * CHUNK_R
        c = (k % nc) * CHUNK_C
        base = (block_base + r * cols + c).astype(_U32)
        noise = _bits_to_normal(_threefry_bits(iota + base))
        x = x_ref[pl.ds(r, CHUNK_R), pl.ds(c, CHUNK_C)]
        coef = c_ref[pl.ds(r, CHUNK_R), :]
        o_ref[pl.ds(r, CHUNK_R), pl.ds(c, CHUNK_C)] = x + coef * (noise - x)
        return carry

    jax.lax.fori_loop(0, nchunks, body, 0)


def kernel(X, steps, mask):
    b, s, d = X.shape
    n_rows = b * s
    ni = 1.0 - jnp.cos(jnp.pi * (1.0 - steps.astype(X.dtype) / MAX_STEPS_) / 2.0)
    coef = jnp.where(mask, (1.0 - ni)[:, None], 0.0).astype(X.dtype)
    coef = coef.reshape(n_rows, 1)
    x2 = X.reshape(n_rows, d)
    grid = n_rows // ROWS_PER_BLOCK
    out = pl.pallas_call(
        _block_body,
        grid=(grid,),
        in_specs=[
            pl.BlockSpec((ROWS_PER_BLOCK, d), lambda i: (i, 0)),
            pl.BlockSpec((ROWS_PER_BLOCK, 1), lambda i: (i, 0)),
        ],
        out_specs=pl.BlockSpec((ROWS_PER_BLOCK, d), lambda i: (i, 0)),
        out_shape=jax.ShapeDtypeStruct((n_rows, d), X.dtype),
        compiler_params=pltpu.CompilerParams(
            dimension_semantics=("parallel",)),
    )(x2, coef)
    return out.reshape(b, s, d)


# speedup vs baseline: 2.5768x; 1.1015x over previous
import jax
import jax.numpy as jnp
import numpy as np
from jax.experimental import pallas as pl
from jax.experimental.pallas import tpu as pltpu

MAX_STEPS_ = 1000
ROWS_PER_BLOCK = 256
ROW_LEN = 2048
CHUNK_R = 8
CHUNK_C = 1024

_U32 = jnp.uint32
_KS1 = 42
_KS2 = 0x1BD11BDA ^ 42

_G = [1.2543749809265137, -0.023982059210538864, 0.45813021063804626,
      -0.28965041041374207, 0.33574575185775757, -0.1841685026884079,
      0.04992347210645676, -0.006709587294608355, 0.0003595015441533178]

_UNIF_LO = np.nextafter(np.float32(-1.0), np.float32(0.0))
_UNIF_SPAN = np.float32(np.float32(1.0) - _UNIF_LO)
_UNIF_OFF = np.float32(_UNIF_LO - _UNIF_SPAN)


def _rotl(x, r):
    return jax.lax.shift_left(x, _U32(r)) | jax.lax.shift_right_logical(x, _U32(32 - r))


def _threefry_bits(x1):
    x0 = x1
    x1 = x0 ^ _rotl(x1, 13)
    for r in (15, 26, 6):
        x0 = x0 + x1
        x1 = x0 ^ _rotl(x1, r)
    x0 = x0 + _U32(_KS1)
    x1 = x1 + _U32(_KS2 + 1)
    for g, rots in ((1, (17, 29, 16, 24)), (2, (13, 15, 26, 6)),
                    (3, (17, 29, 16, 24)), (4, (13, 15, 26, 6))):
        for r in rots:
            x0 = x0 + x1
            x1 = x0 ^ _rotl(x1, r)
        ks = (0, _KS1, _KS2)
        x0 = x0 + _U32(ks[(g + 1) % 3])
        x1 = x1 + _U32((ks[(g + 2) % 3] + g + 1) % (1 << 32))
    return x0 ^ x1


def _bits_to_normal(bits):
    f = jax.lax.bitcast_convert_type(
        jax.lax.shift_right_logical(bits, _U32(9)) | _U32(0x3F800000), jnp.float32)
    u = f * _UNIF_SPAN + _UNIF_OFF
    u = jnp.clip(u, _UNIF_LO, -_UNIF_LO)
    s = 1.0 - u * u
    sq = jnp.sqrt(-jnp.log(s))
    p = jnp.float32(_G[-1])
    for c in _G[-2::-1]:
        p = jnp.float32(c) + p * sq
    return p * u


def _block_body(x_ref, c_ref, o_ref):
    i = pl.program_id(0)
    rows, cols = x_ref.shape
    nc = cols // CHUNK_C
    nchunks = (rows // CHUNK_R) * nc
    iota = (jax.lax.broadcasted_iota(_U32, (CHUNK_R, CHUNK_C), 0) * _U32(cols)
            + jax.lax.broadcasted_iota(_U32, (CHUNK_R, CHUNK_C), 1)
            + _U32(_KS1))
    block_base = i * rows * cols

    def body(k, carry):
        r = (k // nc) * CHUNK_R
        c = (k % nc) * CHUNK_C
        base = (block_base + r * cols + c).astype(_U32)
        noise = _bits_to_normal(_threefry_bits(iota + base))
        x = x_ref[pl.ds(r, CHUNK_R), pl.ds(c, CHUNK_C)]
        coef = c_ref[pl.ds(r, CHUNK_R), :]
        o_ref[pl.ds(r, CHUNK_R), pl.ds(c, CHUNK_C)] = x + coef * (noise - x)
        return carry

    jax.lax.fori_loop(0, nchunks, body, 0)


def kernel(X, steps, mask):
    b, s, d = X.shape
    n_rows = b * s
    ni = 1.0 - jnp.cos(jnp.pi * (1.0 - steps.astype(X.dtype) / MAX_STEPS_) / 2.0)
    coef = jnp.where(mask, (1.0 - ni)[:, None], 0.0).astype(X.dtype)
    coef = coef.reshape(n_rows, 1)
    x2 = X.reshape(n_rows, d)
    grid = n_rows // ROWS_PER_BLOCK
    out = pl.pallas_call(
        _block_body,
        grid=(grid,),
        in_specs=[
            pl.BlockSpec((ROWS_PER_BLOCK, d), lambda i: (i, 0)),
            pl.BlockSpec((ROWS_PER_BLOCK, 1), lambda i: (i, 0)),
        ],
        out_specs=pl.BlockSpec((ROWS_PER_BLOCK, d), lambda i: (i, 0)),
        out_shape=jax.ShapeDtypeStruct((n_rows, d), X.dtype),
        compiler_params=pltpu.CompilerParams(
            dimension_semantics=("parallel",)),
    )(x2, coef)
    return out.reshape(b, s, d)
